# Initial kernel scaffold; baseline (speedup 1.0000x reference)
import jax
import jax.numpy as jnp
from jax.experimental import pallas as pl


def kernel(table, spatial_shape):
    raise NotImplementedError("write your pallas kernel here")


# speedup vs baseline: 2.4297x; 2.4297x over previous
import functools

import jax
import jax.numpy as jnp
from jax import lax
from jax.experimental import pallas as pl
from jax.experimental.pallas import tpu as pltpu
from jax.experimental.pallas import tpu_sc as plsc

D = 384
P = 512
H = 1024
NC = 2
NS = 16
NWJ = 16
NWI = 2
JW = D // NWJ
IW = D // NWI
CH = 64
NCH = IW // CH
L = 16


def _body(table_hbm, out_hbm, jtab, itab, outbuf, sem0, sem1):
    wid = lax.axis_index("s") * NC + lax.axis_index("c")
    jbase = pl.multiple_of((wid % NWJ) * JW, 8)
    ibase = pl.multiple_of((wid // NWJ) * IW, 8)
    sems = (sem0, sem1)

    pltpu.sync_copy(table_hbm.at[pl.ds(jbase, JW)], jtab)

    @pl.loop(0, JW)
    def _init_j(jj):
        @pl.loop(0, P // L, unroll=4)
        def _(k):
            v = jtab[jj, pl.ds(k * L, L)]
            outbuf[0, jj, pl.ds(P + k * L, L)] = v
            outbuf[1, jj, pl.ds(P + k * L, L)] = v

    def fill_i(b, irow):
        @pl.loop(0, P // L, unroll=4)
        def _(k):
            v = itab[irow, pl.ds(k * L, L)]
            for jj in range(JW):
                outbuf[b, jj, pl.ds(k * L, L)] = v

    def start_out(b, i):
        rowstart = pl.multiple_of(i * D + jbase, 8)
        pltpu.async_copy(
            outbuf.at[b], out_hbm.at[pl.ds(rowstart, JW)], sems[b])

    def wait_out(b):
        pltpu.make_async_copy(
            outbuf.at[b], out_hbm.at[pl.ds(jbase, JW)], sems[b]).wait()

    for c in range(NCH):
        cbase = pl.multiple_of(ibase + c * CH, 8)
        pltpu.sync_copy(table_hbm.at[pl.ds(cbase, CH)], itab)
        if c == 0:
            fill_i(0, 0)
            start_out(0, ibase)
            fill_i(1, 1)
            start_out(1, ibase + 1)
            lo = 2
        else:
            lo = 0

        @pl.loop(lo, CH, step=2)
        def _main(ii):
            i = ibase + c * CH + ii
            wait_out(0)
            fill_i(0, ii)
            start_out(0, i)
            wait_out(1)
            fill_i(1, ii + 1)
            start_out(1, i + 1)

    wait_out(0)
    wait_out(1)


@jax.jit
def _positions(table):
    mesh = plsc.VectorSubcoreMesh(
        core_axis_name="c", subcore_axis_name="s",
        num_cores=NC, num_subcores=NS)
    f = pl.kernel(
        _body,
        out_type=jax.ShapeDtypeStruct((D * D, H), jnp.float32),
        mesh=mesh,
        scratch_types=[
            pltpu.VMEM((JW, P), jnp.float32),
            pltpu.VMEM((CH, P), jnp.float32),
            pltpu.VMEM((2, JW, H), jnp.float32),
            pltpu.SemaphoreType.DMA,
            pltpu.SemaphoreType.DMA,
        ],
    )
    return f(table)


def kernel(table, spatial_shape):
    out2d = _positions(table)
    return out2d.reshape(D, D, H)
